# Initial kernel scaffold; baseline (speedup 1.0000x reference)
import jax
import jax.numpy as jnp
from jax.experimental import pallas as pl


def kernel(meta_tensor, emb_tables, np_ln1_g, np_ln1_b, np_w, np_b, np_ln2_g, np_ln2_b, f_ln1_g, f_ln1_b, f_w1, f_b1, f_ln2_g, f_ln2_b, f_w2, f_b2, f_ln3_g, f_ln3_b):
    raise NotImplementedError("write your pallas kernel here")


# speedup vs baseline: 1.1246x; 1.1246x over previous
import functools

import jax
import jax.numpy as jnp
from jax import lax
from jax.experimental import pallas as pl
from jax.experimental.pallas import tpu as pltpu
from jax.experimental.pallas import tpu_sc as plsc

_N_CAT = 26
_VOCAB = 100000
_EMBED = 32
_NUM_CONT = 13

_NC = 2
_NS = 16
_NW = _NC * _NS
_CH = 128
_K = 8


def _sc_gather(table_flat, idx3):
    n_chunks = idx3.shape[1]
    per_w = n_chunks * _CH
    total = _NW * per_w
    mesh = plsc.VectorSubcoreMesh(core_axis_name="c", subcore_axis_name="s")

    @functools.partial(
        pl.kernel,
        mesh=mesh,
        compiler_params=pltpu.CompilerParams(use_tc_tiling_on_sc=False),
        out_type=jax.ShapeDtypeStruct((total, _EMBED), jnp.float32),
        scratch_types=(
            [pltpu.VMEM((n_chunks, _CH), jnp.int32)]
            + [pltpu.VMEM((_CH, _EMBED), jnp.float32) for _ in range(_K)]
            + [pltpu.SemaphoreType.DMA, pltpu.SemaphoreType.DMA]
        ),
    )
    def k(table_hbm, idx_hbm, out_hbm, idx_v, *rest):
        bufs = rest[:_K]
        gsem, wsem = rest[_K], rest[_K + 1]
        wid = lax.axis_index("s") * _NC + lax.axis_index("c")
        base = wid * per_w
        pltpu.sync_copy(idx_hbm.at[wid], idx_v)

        def group(g, carry):
            j0 = g * _K
            for t in range(_K):
                pltpu.async_copy(table_hbm.at[idx_v.at[j0 + t]], bufs[t], gsem)
            for t in range(_K):
                pltpu.make_async_copy(
                    table_hbm.at[idx_v.at[j0 + t]], bufs[t], gsem).wait()
            for t in range(_K):
                pltpu.async_copy(
                    bufs[t], out_hbm.at[pl.ds(base + (j0 + t) * _CH, _CH)], wsem)
            for t in range(_K):
                pltpu.make_async_copy(
                    bufs[t], out_hbm.at[pl.ds(base + (j0 + t) * _CH, _CH)],
                    wsem).wait()
            return carry

        lax.fori_loop(0, n_chunks // _K, group, 0)

    return k(table_flat, idx3)


def _erf(x):
    t = 1.0 / (1.0 + 0.3275911 * jnp.abs(x))
    poly = t * (0.254829592 + t * (-0.284496736 + t * (
        1.421413741 + t * (-1.453152027 + t * 1.061405429))))
    y = 1.0 - poly * jnp.exp(-x * x)
    return jnp.sign(x) * y


def _gelu(x):
    return 0.5 * x * (1.0 + _erf(x * 0.7071067811865476))


def _ln_rows(x, g, b, eps=1e-5):
    m = jnp.mean(x, axis=1, keepdims=True)
    v = jnp.mean((x - m) ** 2, axis=1, keepdims=True)
    return (x - m) * lax.rsqrt(v + eps) * g + b


def _tc_body(xn_ref, cat_ref, g1_ref, b1_ref, npw_ref, npb_ref, g2_ref, b2_ref,
             fg1n_ref, fb1n_ref, fg1c_ref, fb1c_ref, w1a_ref, w1b_ref, fb1_ref,
             fg2_ref, fb2_ref, w2_ref, fb2m_ref, fg3_ref, fb3_ref, out_ref):
    h = _ln_rows(xn_ref[...], g1_ref[...], b1_ref[...])
    h = jnp.dot(h, npw_ref[...], preferred_element_type=jnp.float32) + npb_ref[...]
    h = _gelu(h)
    h = _ln_rows(h, g2_ref[...], b2_ref[...])

    c = cat_ref[...]
    n_tot = h.shape[1] + c.shape[1]
    mu = (jnp.sum(h, axis=1, keepdims=True)
          + jnp.sum(c, axis=1, keepdims=True)) / n_tot
    var = (jnp.sum((h - mu) ** 2, axis=1, keepdims=True)
           + jnp.sum((c - mu) ** 2, axis=1, keepdims=True)) / n_tot
    r = lax.rsqrt(var + 1e-5)
    hn = (h - mu) * r * fg1n_ref[...] + fb1n_ref[...]
    cn = (c - mu) * r * fg1c_ref[...] + fb1c_ref[...]

    y = (jnp.dot(hn, w1a_ref[...], preferred_element_type=jnp.float32)
         + jnp.dot(cn, w1b_ref[...], preferred_element_type=jnp.float32)
         + fb1_ref[...])
    y = _gelu(y)
    y = _ln_rows(y, fg2_ref[...], fb2_ref[...])
    y = jnp.dot(y, w2_ref[...], preferred_element_type=jnp.float32) + fb2m_ref[...]
    y = _gelu(y)
    out_ref[...] = _ln_rows(y, fg3_ref[...], fb3_ref[...])


def _tc_mlp(x_num, cat, params):
    B = x_num.shape[0]
    TB = 512
    grid = (B // TB,)
    row_spec = lambda w: pl.BlockSpec((TB, w), lambda i: (i, 0))
    fixed = lambda a: pl.BlockSpec(a.shape, lambda i: (0,) * a.ndim)
    in_specs = [row_spec(_NUM_CONT), row_spec(_N_CAT * _EMBED)]
    in_specs += [fixed(p) for p in params]
    return pl.pallas_call(
        _tc_body,
        grid=grid,
        in_specs=in_specs,
        out_specs=pl.BlockSpec((TB, 128), lambda i: (i, 0)),
        out_shape=jax.ShapeDtypeStruct((B, 128), jnp.float32),
    )(x_num, cat, *params)


def kernel(meta_tensor, emb_tables, np_ln1_g, np_ln1_b, np_w, np_b,
           np_ln2_g, np_ln2_b, f_ln1_g, f_ln1_b, f_w1, f_b1,
           f_ln2_g, f_ln2_b, f_w2, f_b2, f_ln3_g, f_ln3_b):
    B = meta_tensor.shape[0]
    x_num = meta_tensor[:, :_NUM_CONT]
    x_cat = jnp.clip(meta_tensor[:, _NUM_CONT:].astype(jnp.int32), 0, _VOCAB - 1)
    offs = (jnp.arange(_N_CAT, dtype=jnp.int32) * _VOCAB)[None, :]
    idx = (x_cat + offs).reshape(-1)
    n_chunks = (B * _N_CAT) // (_NW * _CH)
    idx3 = idx.reshape(_NW, n_chunks, _CH)
    table_flat = emb_tables.reshape(_N_CAT * _VOCAB, _EMBED)

    gathered = _sc_gather(table_flat, idx3)
    cat = gathered.reshape(B, _N_CAT * _EMBED)

    row = lambda a: a.reshape(1, -1)
    params = (
        row(np_ln1_g), row(np_ln1_b), np_w, row(np_b),
        row(np_ln2_g), row(np_ln2_b),
        row(f_ln1_g[: 2 * _EMBED]), row(f_ln1_b[: 2 * _EMBED]),
        row(f_ln1_g[2 * _EMBED:]), row(f_ln1_b[2 * _EMBED:]),
        f_w1[: 2 * _EMBED], f_w1[2 * _EMBED:], row(f_b1),
        row(f_ln2_g), row(f_ln2_b), f_w2, row(f_b2),
        row(f_ln3_g), row(f_ln3_b),
    )
    return _tc_mlp(x_num, cat, params)
